# baseline (speedup 1.0000x reference)
import math

import numpy as np
import jax
import jax.numpy as jnp
from jax.experimental import pallas as pl
from jax.experimental.pallas import tpu as pltpu

_L0 = 468
_NL = 75
_NA = 20
_A_REL = list(range(0, 19)) + [54]
_B_REL = list(range(1, 20)) + [55]
_C_REL = list(range(2, 21)) + [56]
_NORM_LM = 17

_LANE0 = (3 * _L0 // 128) * 128
_OFF = 3 * _L0 - _LANE0
_NLANES = 3 * (_L0 + _NL) - _LANE0


def _build_w() -> np.ndarray:
    w = np.zeros((_NLANES, 150 + 4 * _NA), dtype=np.float32)
    for l in range(_NL):
        for ch in range(2):
            w[_OFF + 3 * l + ch, 2 * l + ch] = 1.0
    for i in range(_NA):
        a, b, c = _A_REL[i], _B_REL[i], _C_REL[i]
        for ch in range(2):
            w[_OFF + 3 * a + ch, 150 + 20 * ch + i] += 1.0
            w[_OFF + 3 * b + ch, 150 + 20 * ch + i] -= 1.0
            w[_OFF + 3 * c + ch, 150 + 40 + 20 * ch + i] += 1.0
            w[_OFF + 3 * b + ch, 150 + 40 + 20 * ch + i] -= 1.0
    return w


_W = _build_w()


def _body(xr, x17_ref, w_ref, out_ref, xs_v, sem_xs):
    b = pl.program_id(0)
    cp1 = pltpu.make_async_copy(
        xr.at[b, :, _LANE0:_LANE0 + _NLANES], xs_v, sem_xs)
    cp1.start()

    x17 = x17_ref[0, :, _NORM_LM % 8, :]
    t = x17.shape[0]
    ch3 = jax.lax.broadcasted_iota(jnp.int32, (1, 3), 1)
    m0 = jnp.sum(jnp.where(ch3 == 0, x17, 0.0)) * (1.0 / t)
    m1 = jnp.sum(jnp.where(ch3 == 1, x17, 0.0)) * (1.0 / t)

    cp1.wait()
    xs = xs_v[...]

    c = jnp.dot(xs, w_ref[...], preferred_element_type=jnp.float32)

    g = c[:, :150]
    lane = jax.lax.broadcasted_iota(jnp.int32, (1, 150), 1)
    even = (lane % 2) == 0
    d = g - jnp.where(even, m0, m1)
    dd = d * d
    denom = 1.0 / (t * _NL)
    s0 = jnp.sum(jnp.where(even, dd, 0.0)) * denom
    s1 = jnp.sum(jnp.where(even, 0.0, dd)) * denom
    inv0 = 1.0 / jnp.sqrt(s0)
    inv1 = 1.0 / jnp.sqrt(s1)
    xn = d * jnp.where(even, inv0, inv1)

    dx = jnp.concatenate([xn[1:], xn[t - 1:]], axis=0) - xn

    vax = c[:, 150:170]
    vay = c[:, 170:190]
    vbx = c[:, 190:210]
    vby = c[:, 210:230]
    dot = vax * vbx + vay * vby
    nrm = jnp.sqrt((vax * vax + vay * vay) * (vbx * vbx + vby * vby))
    cos = jnp.clip(dot / nrm, -1.0, 1.0)
    ang = jnp.arctan2(jnp.sqrt(1.0 - cos * cos), cos) * (1.0 / math.pi)

    out = jnp.concatenate([xn, dx, ang], axis=1)
    out = jnp.where(jnp.isnan(out), 0.0, out)
    out_ref[0] = out


def kernel(inputs):
    x = inputs
    batch, t, n, ch = x.shape
    xr = x.reshape(batch, t, n * ch)
    lm_blk = (_NORM_LM // 8) * 8
    return pl.pallas_call(
        _body,
        grid=(batch,),
        in_specs=[
            pl.BlockSpec(memory_space=pl.ANY),
            pl.BlockSpec((1, t, 8, ch), lambda b: (b, 0, lm_blk // 8, 0)),
            pl.BlockSpec((_NLANES, 150 + 4 * _NA), lambda b: (0, 0)),
        ],
        out_specs=pl.BlockSpec((1, t, 320), lambda b: (b, 0, 0)),
        out_shape=jax.ShapeDtypeStruct((batch, t, 320), jnp.float32),
        scratch_shapes=[
            pltpu.VMEM((t, _NLANES), jnp.float32),
            pltpu.SemaphoreType.DMA,
        ],
    )(xr, x, jnp.asarray(_W))


# speedup vs baseline: 11.9245x; 11.9245x over previous
import math

import numpy as np
import jax
import jax.numpy as jnp
from jax.experimental import pallas as pl
from jax.experimental.pallas import tpu as pltpu

_L0 = 468
_NL = 75
_NA = 20
_A_REL = list(range(0, 19)) + [54]
_B_REL = list(range(1, 20)) + [55]
_C_REL = list(range(2, 21)) + [56]
_NORM_LM = 17

_LANE0 = (3 * _L0 // 128) * 128
_OFF = 3 * _L0 - _LANE0
_NLANES = 3 * (_L0 + _NL) - _LANE0


def _build_w() -> np.ndarray:
    w = np.zeros((_NLANES, 150 + 4 * _NA), dtype=np.float32)
    for l in range(_NL):
        for ch in range(2):
            w[_OFF + 3 * l + ch, 2 * l + ch] = 1.0
    for i in range(_NA):
        a, b, c = _A_REL[i], _B_REL[i], _C_REL[i]
        for ch in range(2):
            w[_OFF + 3 * a + ch, 150 + 20 * ch + i] += 1.0
            w[_OFF + 3 * b + ch, 150 + 20 * ch + i] -= 1.0
            w[_OFF + 3 * c + ch, 150 + 40 + 20 * ch + i] += 1.0
            w[_OFF + 3 * b + ch, 150 + 40 + 20 * ch + i] -= 1.0
    return w


_W = _build_w()


def _copies(xr, xs_v, x0_v, sems, idx, slot):
    return (
        pltpu.make_async_copy(
            xr.at[idx, :, _LANE0:_LANE0 + _NLANES], xs_v.at[slot],
            sems.at[slot, 0]),
        pltpu.make_async_copy(
            xr.at[idx, :, 0:128], x0_v.at[slot], sems.at[slot, 1]),
    )


def _body(xr, w_ref, out_ref, xs_v, x0_v, sems):
    b = pl.program_id(0)
    nb = pl.num_programs(0)
    slot = b % 2

    @pl.when(b == 0)
    def _prologue():
        for cp in _copies(xr, xs_v, x0_v, sems, b, slot):
            cp.start()

    @pl.when(b + 1 < nb)
    def _prefetch():
        for cp in _copies(xr, xs_v, x0_v, sems, b + 1, 1 - slot):
            cp.start()

    for cp in _copies(xr, xs_v, x0_v, sems, b, slot):
        cp.wait()

    xs = xs_v[slot]
    x0 = x0_v[slot]
    t = xs.shape[0]

    lane0 = jax.lax.broadcasted_iota(jnp.int32, (1, 128), 1)
    m0 = jnp.sum(jnp.where(lane0 == 3 * _NORM_LM, x0, 0.0)) * (1.0 / t)
    m1 = jnp.sum(jnp.where(lane0 == 3 * _NORM_LM + 1, x0, 0.0)) * (1.0 / t)

    c = jnp.dot(xs, w_ref[...], preferred_element_type=jnp.float32)

    g = c[:, :150]
    lane = jax.lax.broadcasted_iota(jnp.int32, (1, 150), 1)
    even = (lane % 2) == 0
    d = g - jnp.where(even, m0, m1)
    dd = d * d
    denom = 1.0 / (t * _NL)
    s0 = jnp.sum(jnp.where(even, dd, 0.0)) * denom
    s1 = jnp.sum(jnp.where(even, 0.0, dd)) * denom
    inv0 = 1.0 / jnp.sqrt(s0)
    inv1 = 1.0 / jnp.sqrt(s1)
    xn = d * jnp.where(even, inv0, inv1)

    dx = jnp.concatenate([xn[1:], xn[t - 1:]], axis=0) - xn

    vax = c[:, 150:170]
    vay = c[:, 170:190]
    vbx = c[:, 190:210]
    vby = c[:, 210:230]
    dot = vax * vbx + vay * vby
    nrm = jnp.sqrt((vax * vax + vay * vay) * (vbx * vbx + vby * vby))
    cos = jnp.clip(dot / nrm, -1.0, 1.0)
    ang = jnp.arctan2(jnp.sqrt(1.0 - cos * cos), cos) * (1.0 / math.pi)

    out = jnp.concatenate([xn, dx, ang], axis=1)
    out = jnp.where(jnp.isnan(out), 0.0, out)
    out_ref[0] = out


def kernel(inputs):
    x = inputs
    batch, t, n, ch = x.shape
    xr = x.reshape(batch, t, n * ch)
    return pl.pallas_call(
        _body,
        grid=(batch,),
        in_specs=[
            pl.BlockSpec(memory_space=pl.ANY),
            pl.BlockSpec((_NLANES, 150 + 4 * _NA), lambda b: (0, 0)),
        ],
        out_specs=pl.BlockSpec((1, t, 320), lambda b: (b, 0, 0)),
        out_shape=jax.ShapeDtypeStruct((batch, t, 320), jnp.float32),
        scratch_shapes=[
            pltpu.VMEM((2, t, _NLANES), jnp.float32),
            pltpu.VMEM((2, t, 128), jnp.float32),
            pltpu.SemaphoreType.DMA((2, 2)),
        ],
    )(xr, jnp.asarray(_W))


# speedup vs baseline: 12.7564x; 1.0698x over previous
import math

import numpy as np
import jax
import jax.numpy as jnp
from jax.experimental import pallas as pl
from jax.experimental.pallas import tpu as pltpu

_L0 = 468
_NL = 75
_NA = 20
_A_REL = list(range(0, 19)) + [54]
_B_REL = list(range(1, 20)) + [55]
_C_REL = list(range(2, 21)) + [56]
_NORM_LM = 17

_LANE0 = (3 * _L0 // 128) * 128
_OFF = 3 * _L0 - _LANE0
_NLANES = 3 * (_L0 + _NL) - _LANE0
_BPS = 4


def _build_w() -> np.ndarray:
    w = np.zeros((_NLANES, 150 + 4 * _NA), dtype=np.float32)
    for l in range(_NL):
        for ch in range(2):
            w[_OFF + 3 * l + ch, 2 * l + ch] = 1.0
    for i in range(_NA):
        a, b, c = _A_REL[i], _B_REL[i], _C_REL[i]
        for ch in range(2):
            w[_OFF + 3 * a + ch, 150 + 20 * ch + i] += 1.0
            w[_OFF + 3 * b + ch, 150 + 20 * ch + i] -= 1.0
            w[_OFF + 3 * c + ch, 150 + 40 + 20 * ch + i] += 1.0
            w[_OFF + 3 * b + ch, 150 + 40 + 20 * ch + i] -= 1.0
    return w


_W = _build_w()


def _copies(xr, xs_v, x0_v, sems, step, slot):
    b0 = step * _BPS
    return (
        pltpu.make_async_copy(
            xr.at[pl.ds(b0, _BPS), :, _LANE0:_LANE0 + _NLANES],
            xs_v.at[slot], sems.at[slot, 0]),
        pltpu.make_async_copy(
            xr.at[pl.ds(b0, _BPS), :, 0:128], x0_v.at[slot],
            sems.at[slot, 1]),
    )


def _body(xr, w_ref, out_ref, xs_v, x0_v, sems):
    s = pl.program_id(0)
    ns = pl.num_programs(0)
    slot = s % 2

    @pl.when(s == 0)
    def _prologue():
        for cp in _copies(xr, xs_v, x0_v, sems, s, slot):
            cp.start()

    @pl.when(s + 1 < ns)
    def _prefetch():
        for cp in _copies(xr, xs_v, x0_v, sems, s + 1, 1 - slot):
            cp.start()

    for cp in _copies(xr, xs_v, x0_v, sems, s, slot):
        cp.wait()

    t = xs_v.shape[2]
    xs = xs_v[slot].reshape(_BPS * t, _NLANES)
    x0 = x0_v[slot]

    c = jnp.dot(xs, w_ref[...], preferred_element_type=jnp.float32)

    vax = c[:, 150:170]
    vay = c[:, 170:190]
    vbx = c[:, 190:210]
    vby = c[:, 210:230]
    dot = vax * vbx + vay * vby
    nrm = jnp.sqrt((vax * vax + vay * vay) * (vbx * vbx + vby * vby))
    cos = jnp.clip(dot / nrm, -1.0, 1.0)
    ang = jnp.arctan2(jnp.sqrt(1.0 - cos * cos), cos) * (1.0 / math.pi)

    lane0 = jax.lax.broadcasted_iota(jnp.int32, (1, 128), 1)
    lane = jax.lax.broadcasted_iota(jnp.int32, (1, 150), 1)
    even = (lane % 2) == 0
    denom = 1.0 / (t * _NL)
    for i in range(_BPS):
        x17 = x0[i]
        m0 = jnp.sum(jnp.where(lane0 == 3 * _NORM_LM, x17, 0.0)) * (1.0 / t)
        m1 = jnp.sum(
            jnp.where(lane0 == 3 * _NORM_LM + 1, x17, 0.0)) * (1.0 / t)
        g = c[i * t:(i + 1) * t, :150]
        d = g - jnp.where(even, m0, m1)
        dd = d * d
        s0 = jnp.sum(jnp.where(even, dd, 0.0)) * denom
        s1 = jnp.sum(jnp.where(even, 0.0, dd)) * denom
        inv0 = 1.0 / jnp.sqrt(s0)
        inv1 = 1.0 / jnp.sqrt(s1)
        xn = d * jnp.where(even, inv0, inv1)
        dx = jnp.concatenate([xn[1:], xn[t - 1:]], axis=0) - xn
        out = jnp.concatenate([xn, dx, ang[i * t:(i + 1) * t]], axis=1)
        out = jnp.where(jnp.isnan(out), 0.0, out)
        out_ref[i] = out


def kernel(inputs):
    x = inputs
    batch, t, n, ch = x.shape
    xr = x.reshape(batch, t, n * ch)
    return pl.pallas_call(
        _body,
        grid=(batch // _BPS,),
        in_specs=[
            pl.BlockSpec(memory_space=pl.ANY),
            pl.BlockSpec((_NLANES, 150 + 4 * _NA), lambda s: (0, 0)),
        ],
        out_specs=pl.BlockSpec((_BPS, t, 320), lambda s: (s, 0, 0)),
        out_shape=jax.ShapeDtypeStruct((batch, t, 320), jnp.float32),
        scratch_shapes=[
            pltpu.VMEM((2, _BPS, t, _NLANES), jnp.float32),
            pltpu.VMEM((2, _BPS, t, 128), jnp.float32),
            pltpu.SemaphoreType.DMA((2, 2)),
        ],
    )(xr, jnp.asarray(_W))
